# Initial kernel scaffold; baseline (speedup 1.0000x reference)
import jax
import jax.numpy as jnp
from jax.experimental import pallas as pl


def kernel(x_t, v_pred, W):
    raise NotImplementedError("write your pallas kernel here")


# speedup vs baseline: 20.6461x; 20.6461x over previous
import functools

import jax
import jax.numpy as jnp
from jax.experimental import pallas as pl
from jax.experimental.pallas import tpu as pltpu

EPS = 0.01
TOPK = 8


def _ssq_kernel(v_ref, ssq_ref):
    r = pl.program_id(1)
    v = v_ref[0]
    part = jnp.sum(v * v)

    @pl.when(r == 0)
    def _init():
        ssq_ref[...] = jnp.reshape(part, (1, 1, 1))

    @pl.when(r != 0)
    def _acc():
        ssq_ref[...] += jnp.reshape(part, (1, 1, 1))


def _tokens_kernel(x_ref, v_ref, w_ref, c_ref, t_ref, p_ref):
    x = x_ref[0]
    v = v_ref[0]
    w = w_ref[...]
    c = c_ref[...][0]
    a = jnp.dot(x, w, preferred_element_type=jnp.float32)
    xp = x + c * v
    tp = jnp.dot(xp, w, preferred_element_type=jnp.float32)
    t_ref[0] = a / (jnp.sqrt(jnp.sum(a * a, axis=1, keepdims=True)) + 1e-6)
    p_ref[0] = tp / (jnp.sqrt(jnp.sum(tp * tp, axis=1, keepdims=True)) + 1e-6)


def _loss_kernel(t_ref, p_ref, out_ref, *, rb, n):
    b = pl.program_id(0)
    r = pl.program_id(1)

    rows_t = t_ref[0, pl.ds(r * rb, rb), :]
    rows_p = p_ref[0, pl.ds(r * rb, rb), :]
    s_t = jnp.dot(rows_t, t_ref[0].T, preferred_element_type=jnp.float32)
    s_p = jnp.dot(rows_p, p_ref[0].T, preferred_element_type=jnp.float32)

    col = jax.lax.broadcasted_iota(jnp.int32, (rb, n), 1)
    rowg = jax.lax.broadcasted_iota(jnp.int32, (rb, n), 0) + r * rb
    neg = jnp.float32(-jnp.inf)
    work = jnp.where(col == rowg, neg, s_t)
    d = s_p - s_t
    d2 = d * d

    acc = jnp.float32(0.0)
    for _ in range(TOPK):
        m = jnp.max(work, axis=1, keepdims=True)
        sel = work == m
        acc += jnp.sum(jnp.where(sel, d2, 0.0))
        work = jnp.where(sel, neg, work)

    @pl.when((b == 0) & (r == 0))
    def _out_init():
        out_ref[...] = jnp.reshape(acc, (1, 1))

    @pl.when((b != 0) | (r != 0))
    def _out_acc():
        out_ref[...] += jnp.reshape(acc, (1, 1))


def kernel(x_t, v_pred, W):
    bsz, n, d = x_t.shape
    h = W.shape[1]
    ra = 512
    rb = 256

    ssq = pl.pallas_call(
        _ssq_kernel,
        grid=(bsz, n // ra),
        in_specs=[pl.BlockSpec((1, ra, d), lambda b, r: (b, r, 0))],
        out_specs=pl.BlockSpec((1, 1, 1), lambda b, r: (b, 0, 0)),
        out_shape=jax.ShapeDtypeStruct((bsz, 1, 1), jnp.float32),
    )(v_pred)

    rms = jnp.sqrt(ssq / (n * d) + 1e-6)
    c = EPS / rms

    t_hat, p_hat = pl.pallas_call(
        _tokens_kernel,
        grid=(bsz, n // ra),
        in_specs=[
            pl.BlockSpec((1, ra, d), lambda b, r: (b, r, 0)),
            pl.BlockSpec((1, ra, d), lambda b, r: (b, r, 0)),
            pl.BlockSpec((d, h), lambda b, r: (0, 0)),
            pl.BlockSpec((1, 1, 1), lambda b, r: (b, 0, 0)),
        ],
        out_specs=[
            pl.BlockSpec((1, ra, h), lambda b, r: (b, r, 0)),
            pl.BlockSpec((1, ra, h), lambda b, r: (b, r, 0)),
        ],
        out_shape=[
            jax.ShapeDtypeStruct((bsz, n, h), jnp.float32),
            jax.ShapeDtypeStruct((bsz, n, h), jnp.float32),
        ],
    )(x_t, v_pred, W, c)

    out = pl.pallas_call(
        functools.partial(_loss_kernel, rb=rb, n=n),
        grid=(bsz, n // rb),
        in_specs=[
            pl.BlockSpec((1, n, h), lambda b, r: (b, 0, 0)),
            pl.BlockSpec((1, n, h), lambda b, r: (b, 0, 0)),
        ],
        out_specs=pl.BlockSpec((1, 1), lambda b, r: (0, 0)),
        out_shape=jax.ShapeDtypeStruct((1, 1), jnp.float32),
    )(t_hat, p_hat)

    return out[0, 0] / bsz
